# Initial kernel scaffold; baseline (speedup 1.0000x reference)
import jax
import jax.numpy as jnp
from jax.experimental import pallas as pl


def kernel(hidden_states, gate_w, gate_proj_w, up_proj_w, down_proj_w):
    raise NotImplementedError("write your pallas kernel here")


# speedup vs baseline: 1.3109x; 1.3109x over previous
import functools

import jax
import jax.numpy as jnp
from jax import lax
from jax.experimental import pallas as pl
from jax.experimental.pallas import tpu as pltpu
from jax.experimental.pallas import tpu_sc as plsc

_H = 2048
_I = 1408
_E = 8
_K = 2
_T = 4096
_BM = 256
_NPAD = _T * _K + _E * _BM
_NB = _NPAD // _BM
_NC = 2
_NS = 16
_NW = _NC * _NS
_RPW = _NPAD // _NW
_TPW = _T // _NW
_G = 40
_C = 16


def _sc_mesh():
    return plsc.VectorSubcoreMesh(
        core_axis_name="c", subcore_axis_name="s",
        num_cores=_NC, num_subcores=_NS)


def _sc_gather(x2d, tok_pad):
    @functools.partial(
        pl.kernel,
        mesh=_sc_mesh(),
        out_type=jax.ShapeDtypeStruct((_NPAD, _H), jnp.float32),
        scratch_types=[
            pltpu.VMEM((_G,), jnp.int32),
            pltpu.VMEM((_G, _H), jnp.float32),
            pltpu.SemaphoreType.DMA,
        ],
    )
    def k(x_hbm, ids_hbm, out_hbm, idx_v, rows_v, sem):
        wid = lax.axis_index("s") * _NC + lax.axis_index("c")
        base = wid * _RPW
        for c in range(_RPW // _G):
            off = base + c * _G
            pltpu.sync_copy(ids_hbm.at[pl.ds(off, _G)], idx_v)
            pltpu.async_copy(x_hbm.at[idx_v], rows_v, sem).wait()
            pltpu.sync_copy(rows_v, out_hbm.at[pl.ds(off, _G)])

    return k(x2d, tok_pad)


def _tc_gemm_body(be_ref, x_ref, gpw_ref, upw_ref, dpw_ref, wp_ref, y_ref):
    del be_ref
    dn = (((1,), (1,)), ((), ()))
    xb = x_ref[...].astype(jnp.bfloat16)
    gp = lax.dot_general(xb, gpw_ref[0], dn, preferred_element_type=jnp.float32)
    up = lax.dot_general(xb, upw_ref[0], dn, preferred_element_type=jnp.float32)
    act = (gp * jax.nn.sigmoid(gp) * up).astype(jnp.bfloat16)
    y = lax.dot_general(act, dpw_ref[0], dn, preferred_element_type=jnp.float32)
    y_ref[...] = y * wp_ref[...]


def _tc_grouped_gemm(block_expert, x_sorted, gpw, upw, dpw, w_pad):
    grid_spec = pltpu.PrefetchScalarGridSpec(
        num_scalar_prefetch=1,
        grid=(_NB,),
        in_specs=[
            pl.BlockSpec((_BM, _H), lambda i, be: (i, 0)),
            pl.BlockSpec((1, _I, _H), lambda i, be: (be[i], 0, 0)),
            pl.BlockSpec((1, _I, _H), lambda i, be: (be[i], 0, 0)),
            pl.BlockSpec((1, _H, _I), lambda i, be: (be[i], 0, 0)),
            pl.BlockSpec((_BM, 1), lambda i, be: (i, 0)),
        ],
        out_specs=pl.BlockSpec((_BM, _H), lambda i, be: (i, 0)),
    )
    return pl.pallas_call(
        _tc_gemm_body,
        grid_spec=grid_spec,
        out_shape=jax.ShapeDtypeStruct((_NPAD, _H), jnp.float32),
    )(block_expert, x_sorted, gpw, upw, dpw, w_pad)


def _sc_combine(y_sorted, pos0, pos1):
    @functools.partial(
        pl.kernel,
        mesh=_sc_mesh(),
        out_type=jax.ShapeDtypeStruct((_T, _H), jnp.float32),
        scratch_types=[
            pltpu.VMEM((_C,), jnp.int32),
            pltpu.VMEM((_C,), jnp.int32),
            pltpu.VMEM((_C, _H), jnp.float32),
            pltpu.VMEM((_C, _H), jnp.float32),
            pltpu.SemaphoreType.DMA,
            pltpu.SemaphoreType.DMA,
        ],
    )
    def k(y_hbm, p0_hbm, p1_hbm, out_hbm, i0_v, i1_v, a_v, b_v, s0, s1):
        wid = lax.axis_index("s") * _NC + lax.axis_index("c")
        base = wid * _TPW
        for c in range(_TPW // _C):
            off = base + c * _C
            pltpu.sync_copy(p0_hbm.at[pl.ds(off, _C)], i0_v)
            pltpu.sync_copy(p1_hbm.at[pl.ds(off, _C)], i1_v)
            cp0 = pltpu.async_copy(y_hbm.at[i0_v], a_v, s0)
            cp1 = pltpu.async_copy(y_hbm.at[i1_v], b_v, s1)
            cp0.wait()
            cp1.wait()

            def add_col(j, carry):
                for r in range(_C):
                    sl = pl.ds(j * 16, 16)
                    a_v[r, sl] = a_v[r, sl] + b_v[r, sl]
                return carry

            lax.fori_loop(0, _H // 16, add_col, 0)
            pltpu.sync_copy(a_v, out_hbm.at[pl.ds(off, _C)])

    return k(y_sorted, pos0, pos1)


def kernel(hidden_states, gate_w, gate_proj_w, up_proj_w, down_proj_w):
    b, s, h = hidden_states.shape
    x = hidden_states.reshape(-1, h)

    router_logits = x @ gate_w.T
    routing_weights = jax.nn.softmax(router_logits.astype(jnp.float32), axis=1)
    rw_topk, selected_experts = jax.lax.top_k(routing_weights, _K)
    rw_topk = rw_topk / jnp.sum(rw_topk, axis=-1, keepdims=True)

    e_flat = selected_experts.reshape(-1).astype(jnp.int32)
    w_flat = rw_topk.reshape(-1)
    t_flat = jnp.repeat(jnp.arange(_T, dtype=jnp.int32), _K)
    oh = (e_flat[:, None] == jnp.arange(_E, dtype=jnp.int32)[None, :])
    incl = jnp.cumsum(oh.astype(jnp.int32), axis=0)
    counts = incl[-1]
    rank = jnp.take_along_axis(incl, e_flat[:, None], axis=1)[:, 0] - 1
    padded = ((counts + _BM - 1) // _BM) * _BM
    poff = jnp.concatenate(
        [jnp.zeros((1,), jnp.int32), jnp.cumsum(padded).astype(jnp.int32)])
    pos_flat = poff[e_flat] + rank
    tok_pad = jnp.zeros((_NPAD,), jnp.int32).at[pos_flat].set(t_flat)
    w_pad = jnp.zeros((_NPAD,), jnp.float32).at[pos_flat].set(w_flat)
    blk_start = jnp.arange(_NB, dtype=jnp.int32) * _BM
    block_expert = jnp.minimum(
        jnp.sum((blk_start[:, None] >= poff[None, 1:]).astype(jnp.int32), axis=1),
        _E - 1).astype(jnp.int32)
    pos2 = pos_flat.reshape(_T, _K)

    x_sorted = _sc_gather(x, tok_pad)
    y_sorted = _tc_grouped_gemm(
        block_expert, x_sorted,
        gate_proj_w.astype(jnp.bfloat16),
        up_proj_w.astype(jnp.bfloat16),
        down_proj_w.astype(jnp.bfloat16),
        w_pad.reshape(_NPAD, 1))
    out = _sc_combine(y_sorted, pos2[:, 0], pos2[:, 1])
    return out.reshape(b, s, h)
